# initial kernel scaffold (unmeasured)
import jax
import jax.numpy as jnp
from jax import lax
from jax.experimental import pallas as pl
from jax.experimental.pallas import tpu as pltpu

N_DEV = 4
B, Sq, Skv = 2, 256, 512
Dh = 64
H_LOC = 8
DQ_LOC = H_LOC * Dh
D = 768
SCALE = 0.125


def kernel(x, Wq, Wo, K_ext, V_ext):
    i = lax.axis_index("i")
    K_loc = lax.dynamic_slice_in_dim(K_ext, i * H_LOC, H_LOC, axis=2)
    V_loc = lax.dynamic_slice_in_dim(V_ext, i * H_LOC, H_LOC, axis=2)
    K_loc = K_loc.reshape(B, Skv, DQ_LOC)
    V_loc = V_loc.reshape(B, Skv, DQ_LOC)

    def body(x_ref, wq_ref, wo_ref, k_ref, v_ref, out_ref,
             acc_ref, comm_ref, send_sems, recv_sems):
        my = lax.axis_index("i")
        right = lax.rem(my + 1, N_DEV)

        for b in range(B):
            q_b = jnp.dot(x_ref[b], wq_ref[...],
                          preferred_element_type=jnp.float32)
            for h in range(H_LOC):
                qh = q_b[:, h * Dh:(h + 1) * Dh]
                kh = k_ref[b, :, h * Dh:(h + 1) * Dh]
                vh = v_ref[b, :, h * Dh:(h + 1) * Dh]
                s = lax.dot_general(
                    qh, kh, (((1,), (1,)), ((), ())),
                    preferred_element_type=jnp.float32) * SCALE
                m = jnp.max(s, axis=1, keepdims=True)
                p = jnp.exp(s - m)
                l = jnp.sum(p, axis=1, keepdims=True)
                o = jnp.dot(p, vh, preferred_element_type=jnp.float32) / l
                acc_ref[b, :, h * Dh:(h + 1) * Dh] = o
            comm_ref[0, b] = jnp.dot(acc_ref[b], wo_ref[...],
                                     preferred_element_type=jnp.float32)
        out_ref[...] = comm_ref[0]

        for hop in range(N_DEV - 1):
            rdma = pltpu.make_async_remote_copy(
                src_ref=comm_ref.at[hop],
                dst_ref=comm_ref.at[hop + 1],
                send_sem=send_sems.at[hop],
                recv_sem=recv_sems.at[hop],
                device_id=(right,),
                device_id_type=pl.DeviceIdType.MESH,
            )
            rdma.start()
            rdma.wait()
            out_ref[...] += comm_ref[hop + 1]

    return pl.pallas_call(
        body,
        out_shape=jax.ShapeDtypeStruct((B, Sq, D), jnp.float32),
        in_specs=[pl.BlockSpec(memory_space=pltpu.VMEM)] * 5,
        out_specs=pl.BlockSpec(memory_space=pltpu.VMEM),
        scratch_shapes=[
            pltpu.VMEM((B, Sq, DQ_LOC), jnp.float32),
            pltpu.VMEM((N_DEV, B, Sq, D), jnp.float32),
            pltpu.SemaphoreType.DMA((N_DEV - 1,)),
            pltpu.SemaphoreType.DMA((N_DEV - 1,)),
        ],
        compiler_params=pltpu.CompilerParams(collective_id=0),
    )(x, Wq, Wo, K_loc, V_loc)


# baseline (device time: 75551 ns/iter reference)
import jax
import jax.numpy as jnp
from jax import lax
from jax.experimental import pallas as pl
from jax.experimental.pallas import tpu as pltpu

N_DEV = 4
B, Sq, Skv = 2, 256, 512
Dh = 64
H_LOC = 8
DQ_LOC = H_LOC * Dh
D = 768
SCALE = 0.125


def kernel(x, Wq, Wo, K_ext, V_ext):
    i = lax.axis_index("i")
    K_loc = lax.dynamic_slice_in_dim(K_ext, i * H_LOC, H_LOC, axis=2)
    V_loc = lax.dynamic_slice_in_dim(V_ext, i * H_LOC, H_LOC, axis=2)
    K_loc = K_loc.reshape(B, Skv, DQ_LOC)
    V_loc = V_loc.reshape(B, Skv, DQ_LOC)

    def body(x_ref, wq_ref, wo_ref, k_ref, v_ref, out_ref,
             acc_ref, comm_ref, send_sems, recv_sems):
        my = lax.axis_index("i")
        right = lax.rem(my + 1, N_DEV)
        left = lax.rem(my + N_DEV - 1, N_DEV)

        barrier_sem = pltpu.get_barrier_semaphore()
        for nbr in (left, right):
            pl.semaphore_signal(barrier_sem, inc=1, device_id=(nbr,),
                                device_id_type=pl.DeviceIdType.MESH)
        pl.semaphore_wait(barrier_sem, 2)

        for b in range(B):
            q_b = jnp.dot(x_ref[b], wq_ref[...],
                          preferred_element_type=jnp.float32)
            for h in range(H_LOC):
                qh = q_b[:, h * Dh:(h + 1) * Dh]
                kh = k_ref[b, :, h * Dh:(h + 1) * Dh]
                vh = v_ref[b, :, h * Dh:(h + 1) * Dh]
                s = lax.dot_general(
                    qh, kh, (((1,), (1,)), ((), ())),
                    preferred_element_type=jnp.float32) * SCALE
                m = jnp.max(s, axis=1, keepdims=True)
                p = jnp.exp(s - m)
                l = jnp.sum(p, axis=1, keepdims=True)
                o = jnp.dot(p, vh, preferred_element_type=jnp.float32) / l
                acc_ref[b, :, h * Dh:(h + 1) * Dh] = o
            comm_ref[0, b] = jnp.dot(acc_ref[b], wo_ref[...],
                                     preferred_element_type=jnp.float32)
        out_ref[...] = comm_ref[0]

        for hop in range(N_DEV - 1):
            rdma = pltpu.make_async_remote_copy(
                src_ref=comm_ref.at[hop],
                dst_ref=comm_ref.at[hop + 1],
                send_sem=send_sems.at[hop],
                recv_sem=recv_sems.at[hop],
                device_id=(right,),
                device_id_type=pl.DeviceIdType.MESH,
            )
            rdma.start()
            rdma.wait()
            out_ref[...] += comm_ref[hop + 1]

    return pl.pallas_call(
        body,
        out_shape=jax.ShapeDtypeStruct((B, Sq, D), jnp.float32),
        in_specs=[pl.BlockSpec(memory_space=pltpu.VMEM)] * 5,
        out_specs=pl.BlockSpec(memory_space=pltpu.VMEM),
        scratch_shapes=[
            pltpu.VMEM((B, Sq, DQ_LOC), jnp.float32),
            pltpu.VMEM((N_DEV, B, Sq, D), jnp.float32),
            pltpu.SemaphoreType.DMA((N_DEV - 1,)),
            pltpu.SemaphoreType.DMA((N_DEV - 1,)),
        ],
        compiler_params=pltpu.CompilerParams(collective_id=0),
    )(x, Wq, Wo, K_loc, V_loc)


# device time: 35344 ns/iter; 2.1376x vs baseline; 2.1376x over previous
import jax
import jax.numpy as jnp
from jax import lax
from jax.experimental import pallas as pl
from jax.experimental.pallas import tpu as pltpu

N_DEV = 4
B, Sq, Skv = 2, 256, 512
Dh = 64
H_LOC = 8
DQ_LOC = H_LOC * Dh
D = 768
R = B * Sq
RC = R // N_DEV
SCALE = 0.125


def kernel(x, Wq, Wo, K_ext, V_ext):
    i = lax.axis_index("i")
    K_loc = lax.dynamic_slice_in_dim(K_ext, i * H_LOC, H_LOC, axis=2)
    V_loc = lax.dynamic_slice_in_dim(V_ext, i * H_LOC, H_LOC, axis=2)
    K_loc = K_loc.reshape(B, Skv, DQ_LOC)
    V_loc = V_loc.reshape(B, Skv, DQ_LOC)

    def body(x_ref, wq_ref, wo_ref, k_ref, v_ref, out_ref,
             acc_ref, p_ref, sbf_ref, rbf_ref, send_sems, recv_sems):
        my = lax.axis_index("i")
        pa = my ^ 1
        pb = my ^ 3
        pa3 = my ^ 2

        barrier_sem = pltpu.get_barrier_semaphore()
        for nbr in (pa, pb):
            pl.semaphore_signal(barrier_sem, inc=1, device_id=(nbr,),
                                device_id_type=pl.DeviceIdType.MESH)
        pl.semaphore_wait(barrier_sem, 2)

        for b in range(B):
            q_b = jnp.dot(x_ref[b], wq_ref[...],
                          preferred_element_type=jnp.float32)
            for h in range(H_LOC):
                qh = q_b[:, h * Dh:(h + 1) * Dh]
                kh = k_ref[b, :, h * Dh:(h + 1) * Dh]
                vh = v_ref[b, :, h * Dh:(h + 1) * Dh]
                s = lax.dot_general(
                    qh, kh, (((1,), (1,)), ((), ())),
                    preferred_element_type=jnp.float32) * SCALE
                m = jnp.max(s, axis=1, keepdims=True)
                p = jnp.exp(s - m)
                l = jnp.sum(p, axis=1, keepdims=True)
                o = jnp.dot(p, vh, preferred_element_type=jnp.float32) / l
                acc_ref[2 * b, :, h * Dh:(h + 1) * Dh] = o[:RC]
                acc_ref[2 * b + 1, :, h * Dh:(h + 1) * Dh] = o[RC:]

        rdmas = []
        for k, c in enumerate((pa, pa3, my, pb)):
            g = jnp.dot(acc_ref[c], wo_ref[...],
                        preferred_element_type=jnp.float32)
            p_ref[c] = g
            if k < 2:
                sbf_ref[c] = g.astype(jnp.bfloat16)
                rdma = pltpu.make_async_remote_copy(
                    src_ref=sbf_ref.at[c],
                    dst_ref=rbf_ref.at[k],
                    send_sem=send_sems.at[k],
                    recv_sem=recv_sems.at[k],
                    device_id=(pa,),
                    device_id_type=pl.DeviceIdType.MESH,
                )
                rdma.start()
                rdmas.append(rdma)
        rdmas[0].wait()
        rdmas[1].wait()
        p_ref[my] = p_ref[my] + rbf_ref[0].astype(jnp.float32)

        g3 = p_ref[pb] + rbf_ref[1].astype(jnp.float32)
        sbf_ref[pb] = g3.astype(jnp.bfloat16)
        rdma2 = pltpu.make_async_remote_copy(
            src_ref=sbf_ref.at[pb], dst_ref=rbf_ref.at[2],
            send_sem=send_sems.at[2], recv_sem=recv_sems.at[2],
            device_id=(pb,), device_id_type=pl.DeviceIdType.MESH,
        )
        rdma2.start()
        rdma2.wait()
        red = p_ref[my] + rbf_ref[2].astype(jnp.float32)
        out_ref[my] = red
        sbf_ref[my] = red.astype(jnp.bfloat16)

        rdma3 = pltpu.make_async_remote_copy(
            src_ref=sbf_ref.at[my], dst_ref=rbf_ref.at[3],
            send_sem=send_sems.at[3], recv_sem=recv_sems.at[3],
            device_id=(pb,), device_id_type=pl.DeviceIdType.MESH,
        )
        rdma3.start()
        rdma3.wait()
        out_ref[pb] = rbf_ref[3].astype(jnp.float32)

        rdma4 = pltpu.make_async_remote_copy(
            src_ref=sbf_ref.at[my], dst_ref=rbf_ref.at[4],
            send_sem=send_sems.at[4], recv_sem=recv_sems.at[4],
            device_id=(pa,), device_id_type=pl.DeviceIdType.MESH,
        )
        rdma5 = pltpu.make_async_remote_copy(
            src_ref=rbf_ref.at[3], dst_ref=rbf_ref.at[5],
            send_sem=send_sems.at[5], recv_sem=recv_sems.at[5],
            device_id=(pa,), device_id_type=pl.DeviceIdType.MESH,
        )
        rdma4.start()
        rdma5.start()
        rdma4.wait()
        rdma5.wait()
        out_ref[pa] = rbf_ref[4].astype(jnp.float32)
        out_ref[pa3] = rbf_ref[5].astype(jnp.float32)

    out = pl.pallas_call(
        body,
        out_shape=jax.ShapeDtypeStruct((N_DEV, RC, D), jnp.float32),
        in_specs=[pl.BlockSpec(memory_space=pltpu.VMEM)] * 5,
        out_specs=pl.BlockSpec(memory_space=pltpu.VMEM),
        scratch_shapes=[
            pltpu.VMEM((N_DEV, RC, DQ_LOC), jnp.float32),
            pltpu.VMEM((N_DEV, RC, D), jnp.float32),
            pltpu.VMEM((N_DEV, RC, D), jnp.bfloat16),
            pltpu.VMEM((6, RC, D), jnp.bfloat16),
            pltpu.SemaphoreType.DMA((6,)),
            pltpu.SemaphoreType.DMA((6,)),
        ],
        compiler_params=pltpu.CompilerParams(collective_id=0),
    )(x, Wq, Wo, K_loc, V_loc)
    return out.reshape(B, Sq, D)
